# baseline (speedup 1.0000x reference)
import functools

import jax
import jax.numpy as jnp
from jax import lax
from jax.experimental import pallas as pl
from jax.experimental.pallas import tpu as pltpu
from jax.experimental.pallas import tpu_sc as plsc

N = 10000
E = 320000
D = 128
DP = 144
NC, NS = 2, 16
RPT = N // NS
CW = 80


def _segsum_body(table, eidx, out, acc, gidx, sidx, rows, sem_g, sem_s,
                 *, dp, nb, cw, g):
    chunks = E // (NS * cw)
    ng = chunks // g
    c = lax.axis_index("c")
    s = lax.axis_index("s")
    base = s * chunks

    def zstore(t, _):
        r = t // (dp // 16)
        j = t % (dp // 16)
        rows[0, r, pl.ds(j * 16, 16)] = jnp.zeros((16,), jnp.float32)
        return 0
    lax.fori_loop(0, cw * (dp // 16), zstore, 0)
    for k in range(RPT // cw):
        pltpu.sync_copy(rows.at[0, pl.ds(0, cw)],
                        acc.at[pl.ds(s * RPT + k * cw, cw)])
    if RPT % cw:
        pltpu.sync_copy(rows.at[0, pl.ds(0, RPT % cw)],
                        acc.at[pl.ds(s * RPT + (RPT // cw) * cw, RPT % cw)])

    pltpu.sync_copy(eidx.at[c, pl.ds(base, g)], gidx.at[0])
    pltpu.sync_copy(eidx.at[1 - c, pl.ds(base, g)], sidx.at[0])
    for j in range(nb):
        pltpu.async_copy(table.at[gidx.at[0, j]], rows.at[j], sem_g)

    plsc.subcore_barrier()

    def step(i, _):
        grp = i // g
        pos = i % g
        slot = grp % 2
        buf = i % nb

        pltpu.make_async_copy(table.at[gidx.at[slot, pos]], rows.at[buf],
                              sem_g).wait()

        @pl.when(i < chunks - 1)
        def _():
            pltpu.async_copy(rows.at[buf], acc.at[sidx.at[slot, pos]], sem_s,
                             add=True)

        @pl.when(i == chunks - 1)
        def _():
            pltpu.sync_copy(rows.at[buf], acc.at[sidx.at[slot, pos]], add=True)

        @pl.when(i > 0)
        def _():
            pltpu.make_async_copy(
                rows.at[(i - 1) % nb],
                acc.at[sidx.at[((i - 1) // g) % 2, (i - 1) % g]], sem_s).wait()

        @pl.when(jnp.logical_and(pos == 0, grp + 1 < ng))
        def _():
            nbase = base + (grp + 1) * g
            pltpu.sync_copy(eidx.at[c, pl.ds(nbase, g)], gidx.at[(grp + 1) % 2])
            pltpu.sync_copy(eidx.at[1 - c, pl.ds(nbase, g)], sidx.at[(grp + 1) % 2])

        nxt = i + nb - 1

        @pl.when(jnp.logical_and(i > 0, nxt < chunks))
        def _():
            pltpu.async_copy(table.at[gidx.at[(nxt // g) % 2, nxt % g]],
                             rows.at[(i - 1) % nb], sem_g)
        return 0
    lax.fori_loop(0, chunks, step, 0)

    plsc.subcore_barrier()

    pltpu.sync_copy(acc.at[pl.ds(s * RPT, RPT)],
                    out.at[c, pl.ds(s * RPT, RPT)])


@functools.partial(jax.jit, static_argnames=("dp", "nb", "cw", "g"))
def _segsum(table, eidx, *, dp, nb, cw, g):
    mesh = plsc.VectorSubcoreMesh(core_axis_name="c", subcore_axis_name="s",
                                  num_cores=NC, num_subcores=NS)
    f = pl.kernel(
        functools.partial(_segsum_body, dp=dp, nb=nb, cw=cw, g=g),
        out_type=jax.ShapeDtypeStruct((2, N, dp), jnp.float32),
        mesh=mesh,
        scratch_types=[
            pltpu.VMEM_SHARED((N, dp), jnp.float32),
            pltpu.VMEM((2, g, cw), jnp.int32),
            pltpu.VMEM((2, g, cw), jnp.int32),
            pltpu.VMEM((nb, cw, dp), jnp.float32),
            pltpu.SemaphoreType.DMA,
            pltpu.SemaphoreType.DMA,
        ],
        compiler_params=pltpu.CompilerParams(use_tc_tiling_on_sc=False),
    )
    return f(table, eidx)


def _dense_body(sums_f, sums_b, cnt_f, cnt_b, xin, wl, wlt, wr, wrt, b, bt,
                out, *, dp_s):
    sf = sums_f[0][:, :D]
    sb = sums_b[0][:, :D]
    mf = sf / jnp.maximum(cnt_f[0][:, D:D + 1], 1.0)
    mb = sb / jnp.maximum(cnt_b[0][:, D:D + 1], 1.0)
    xs = xin[...]
    w_self = wr[...] + wrt[...]
    out[...] = (jnp.dot(mf, wl[...], preferred_element_type=jnp.float32)
                + jnp.dot(mb, wlt[...], preferred_element_type=jnp.float32)
                + jnp.dot(xs, w_self, preferred_element_type=jnp.float32)
                + b[...] + bt[...])


@functools.partial(jax.jit, static_argnames=("dp_s",))
def _dense(sums, cnts, xin, wl, wlt, wr, wrt, b, bt, *, dp_s):
    blk = 1000
    grid = (N // blk,)
    return pl.pallas_call(
        functools.partial(_dense_body, dp_s=dp_s),
        grid=grid,
        in_specs=[
            pl.BlockSpec((1, blk, dp_s), lambda i: (0, i, 0)),
            pl.BlockSpec((1, blk, dp_s), lambda i: (1, i, 0)),
            pl.BlockSpec((1, blk, DP), lambda i: (0, i, 0)),
            pl.BlockSpec((1, blk, DP), lambda i: (1, i, 0)),
            pl.BlockSpec((blk, D), lambda i: (i, 0)),
            pl.BlockSpec((D, D), lambda i: (0, 0)),
            pl.BlockSpec((D, D), lambda i: (0, 0)),
            pl.BlockSpec((D, D), lambda i: (0, 0)),
            pl.BlockSpec((D, D), lambda i: (0, 0)),
            pl.BlockSpec((1, D), lambda i: (0, 0)),
            pl.BlockSpec((1, D), lambda i: (0, 0)),
        ],
        out_specs=pl.BlockSpec((blk, D), lambda i: (i, 0)),
        out_shape=jax.ShapeDtypeStruct((N, D), jnp.float32),
    )(sums, sums, cnts, cnts, xin, wl, wlt, wr, wrt, b, bt)


def kernel(x, edge_index, Wl_1, Wr_1, b_1, Wl_1T, Wr_1T, b_1T,
           Wl_2, Wr_2, b_2, Wl_2T, Wr_2T, b_2T):
    ei32 = edge_index.astype(jnp.int32)
    ei = ei32.reshape(2, E // CW, CW)
    ei2 = ei32.reshape(2, E // 160, 160)
    x_aug = jnp.concatenate(
        [x, jnp.ones((N, 1), jnp.float32),
         jnp.zeros((N, DP - D - 1), jnp.float32)], axis=1)
    b_1 = b_1.reshape(1, D)
    b_1T = b_1T.reshape(1, D)
    b_2 = b_2.reshape(1, D)
    b_2T = b_2T.reshape(1, D)

    sums1 = _segsum(x_aug, ei, dp=DP, nb=3, cw=CW, g=10)
    h = _dense(sums1, sums1, x, Wl_1, Wl_1T, Wr_1, Wr_1T, b_1, b_1T, dp_s=DP)
    sums2 = _segsum(h, ei2, dp=D, nb=2, cw=160, g=5)
    out = _dense(sums2, sums1, h, Wl_2, Wl_2T, Wr_2, Wr_2T, b_2, b_2T, dp_s=D)
    return out


# speedup vs baseline: 1.2218x; 1.2218x over previous
import functools

import jax
import jax.numpy as jnp
from jax import lax
from jax.experimental import pallas as pl
from jax.experimental.pallas import tpu as pltpu
from jax.experimental.pallas import tpu_sc as plsc

N = 10000
E = 320000
D = 128
DP = 144
NC, NS = 2, 16
RPT = N // NS
CW = 80


def _segsum_body(table, eidx, out, acc, gidx, sidx, rows, sem_g, sem_s,
                 sem_i, *, dp, nb, cw, g):
    chunks = E // (NS * cw)
    ng = chunks // g
    c = lax.axis_index("c")
    s = lax.axis_index("s")
    base = s * chunks

    def zstore(t, _):
        r = t // (dp // 16)
        j = t % (dp // 16)
        rows[0, r, pl.ds(j * 16, 16)] = jnp.zeros((16,), jnp.float32)
        return 0
    lax.fori_loop(0, cw * (dp // 16), zstore, 0)
    for k in range(RPT // cw):
        pltpu.sync_copy(rows.at[0, pl.ds(0, cw)],
                        acc.at[pl.ds(s * RPT + k * cw, cw)])
    if RPT % cw:
        pltpu.sync_copy(rows.at[0, pl.ds(0, RPT % cw)],
                        acc.at[pl.ds(s * RPT + (RPT // cw) * cw, RPT % cw)])

    pltpu.sync_copy(eidx.at[c, pl.ds(base, g)], gidx.at[0])
    pltpu.sync_copy(eidx.at[1 - c, pl.ds(base, g)], sidx.at[0])
    for j in range(nb):
        pltpu.async_copy(table.at[gidx.at[0, j]], rows.at[j], sem_g)
    if E // (NS * cw) > g:
        pltpu.async_copy(eidx.at[c, pl.ds(base + g, g)], gidx.at[1], sem_i)
        pltpu.async_copy(eidx.at[1 - c, pl.ds(base + g, g)], sidx.at[1], sem_i)

    plsc.subcore_barrier()

    def step(i, _):
        grp = i // g
        pos = i % g
        slot = grp % 2
        buf = i % nb

        @pl.when(jnp.logical_and(pos == g - nb + 1, grp + 1 < ng))
        def _():
            nslot = (grp + 1) % 2
            pltpu.make_async_copy(eidx.at[c, pl.ds(base, g)],
                                  gidx.at[nslot], sem_i).wait()
            pltpu.make_async_copy(eidx.at[1 - c, pl.ds(base, g)],
                                  sidx.at[nslot], sem_i).wait()

        pltpu.make_async_copy(table.at[gidx.at[slot, pos]], rows.at[buf],
                              sem_g).wait()

        @pl.when(i < chunks - 1)
        def _():
            pltpu.async_copy(rows.at[buf], acc.at[sidx.at[slot, pos]], sem_s,
                             add=True)

        @pl.when(i == chunks - 1)
        def _():
            pltpu.sync_copy(rows.at[buf], acc.at[sidx.at[slot, pos]], add=True)

        @pl.when(i > 0)
        def _():
            pltpu.make_async_copy(
                rows.at[(i - 1) % nb],
                acc.at[sidx.at[((i - 1) // g) % 2, (i - 1) % g]], sem_s).wait()

        @pl.when(jnp.logical_and(pos == 0,
                                 jnp.logical_and(grp > 0, grp + 1 < ng)))
        def _():
            nbase = base + (grp + 1) * g
            pltpu.async_copy(eidx.at[c, pl.ds(nbase, g)],
                             gidx.at[(grp + 1) % 2], sem_i)
            pltpu.async_copy(eidx.at[1 - c, pl.ds(nbase, g)],
                             sidx.at[(grp + 1) % 2], sem_i)

        nxt = i + nb - 1

        @pl.when(jnp.logical_and(i > 0, nxt < chunks))
        def _():
            pltpu.async_copy(table.at[gidx.at[(nxt // g) % 2, nxt % g]],
                             rows.at[(i - 1) % nb], sem_g)
        return 0
    lax.fori_loop(0, chunks, step, 0)

    plsc.subcore_barrier()

    pltpu.sync_copy(acc.at[pl.ds(s * RPT, RPT)],
                    out.at[c, pl.ds(s * RPT, RPT)])


@functools.partial(jax.jit, static_argnames=("dp", "nb", "cw", "g"))
def _segsum(table, eidx, *, dp, nb, cw, g):
    mesh = plsc.VectorSubcoreMesh(core_axis_name="c", subcore_axis_name="s",
                                  num_cores=NC, num_subcores=NS)
    f = pl.kernel(
        functools.partial(_segsum_body, dp=dp, nb=nb, cw=cw, g=g),
        out_type=jax.ShapeDtypeStruct((2, N, dp), jnp.float32),
        mesh=mesh,
        scratch_types=[
            pltpu.VMEM_SHARED((N, dp), jnp.float32),
            pltpu.VMEM((2, g, cw), jnp.int32),
            pltpu.VMEM((2, g, cw), jnp.int32),
            pltpu.VMEM((nb, cw, dp), jnp.float32),
            pltpu.SemaphoreType.DMA,
            pltpu.SemaphoreType.DMA,
            pltpu.SemaphoreType.DMA,
        ],
        compiler_params=pltpu.CompilerParams(use_tc_tiling_on_sc=False),
    )
    return f(table, eidx)


def _dense_body(sums_f, sums_b, cnt_f, cnt_b, xin, wl, wlt, wr, wrt, b, bt,
                out, *, dp_s):
    sf = sums_f[0][:, :D]
    sb = sums_b[0][:, :D]
    mf = sf / jnp.maximum(cnt_f[0][:, D:D + 1], 1.0)
    mb = sb / jnp.maximum(cnt_b[0][:, D:D + 1], 1.0)
    xs = xin[...]
    w_self = wr[...] + wrt[...]
    out[...] = (jnp.dot(mf, wl[...], preferred_element_type=jnp.float32)
                + jnp.dot(mb, wlt[...], preferred_element_type=jnp.float32)
                + jnp.dot(xs, w_self, preferred_element_type=jnp.float32)
                + b[...] + bt[...])


@functools.partial(jax.jit, static_argnames=("dp_s",))
def _dense(sums, cnts, xin, wl, wlt, wr, wrt, b, bt, *, dp_s):
    blk = 1000
    grid = (N // blk,)
    return pl.pallas_call(
        functools.partial(_dense_body, dp_s=dp_s),
        grid=grid,
        in_specs=[
            pl.BlockSpec((1, blk, dp_s), lambda i: (0, i, 0)),
            pl.BlockSpec((1, blk, dp_s), lambda i: (1, i, 0)),
            pl.BlockSpec((1, blk, DP), lambda i: (0, i, 0)),
            pl.BlockSpec((1, blk, DP), lambda i: (1, i, 0)),
            pl.BlockSpec((blk, D), lambda i: (i, 0)),
            pl.BlockSpec((D, D), lambda i: (0, 0)),
            pl.BlockSpec((D, D), lambda i: (0, 0)),
            pl.BlockSpec((D, D), lambda i: (0, 0)),
            pl.BlockSpec((D, D), lambda i: (0, 0)),
            pl.BlockSpec((1, D), lambda i: (0, 0)),
            pl.BlockSpec((1, D), lambda i: (0, 0)),
        ],
        out_specs=pl.BlockSpec((blk, D), lambda i: (i, 0)),
        out_shape=jax.ShapeDtypeStruct((N, D), jnp.float32),
    )(sums, sums, cnts, cnts, xin, wl, wlt, wr, wrt, b, bt)


def kernel(x, edge_index, Wl_1, Wr_1, b_1, Wl_1T, Wr_1T, b_1T,
           Wl_2, Wr_2, b_2, Wl_2T, Wr_2T, b_2T):
    ei = edge_index.astype(jnp.int32).reshape(2, E // CW, CW)
    x_aug = jnp.concatenate(
        [x, jnp.ones((N, 1), jnp.float32),
         jnp.zeros((N, DP - D - 1), jnp.float32)], axis=1)
    b_1 = b_1.reshape(1, D)
    b_1T = b_1T.reshape(1, D)
    b_2 = b_2.reshape(1, D)
    b_2T = b_2T.reshape(1, D)

    sums1 = _segsum(x_aug, ei, dp=DP, nb=3, cw=CW, g=10)
    h = _dense(sums1, sums1, x, Wl_1, Wl_1T, Wr_1, Wr_1T, b_1, b_1T, dp_s=DP)
    sums2 = _segsum(h, ei, dp=D, nb=4, cw=CW, g=10)
    out = _dense(sums2, sums1, h, Wl_2, Wl_2T, Wr_2, Wr_2T, b_2, b_2T, dp_s=D)
    return out


# speedup vs baseline: 1.4239x; 1.1654x over previous
import functools

import jax
import jax.numpy as jnp
from jax import lax
from jax.experimental import pallas as pl
from jax.experimental.pallas import tpu as pltpu
from jax.experimental.pallas import tpu_sc as plsc

N = 10000
E = 320000
D = 128
DP = 144
NC, NS = 2, 16
RPT = N // NS
CW = 80


def _segsum_body(table, eidx, out, acc, gidx, sidx, rows, sem_g, sem_s,
                 sem_i, *, dp, nb, cw, g):
    chunks = E // (NS * cw)
    ng = chunks // g
    c = lax.axis_index("c")
    s = lax.axis_index("s")
    base = s * chunks

    def zstore(t, _):
        r = t // (dp // 16)
        j = t % (dp // 16)
        rows[0, r, pl.ds(j * 16, 16)] = jnp.zeros((16,), jnp.float32)
        return 0
    lax.fori_loop(0, cw * (dp // 16), zstore, 0)
    for k in range(RPT // cw):
        pltpu.sync_copy(rows.at[0, pl.ds(0, cw)],
                        acc.at[pl.ds(s * RPT + k * cw, cw)])
    if RPT % cw:
        pltpu.sync_copy(rows.at[0, pl.ds(0, RPT % cw)],
                        acc.at[pl.ds(s * RPT + (RPT // cw) * cw, RPT % cw)])

    pltpu.sync_copy(eidx.at[c, pl.ds(base, g)], gidx.at[0])
    pltpu.sync_copy(eidx.at[1 - c, pl.ds(base, g)], sidx.at[0])
    for j in range(nb):
        pltpu.async_copy(table.at[gidx.at[0, j]], rows.at[j], sem_g)
    if E // (NS * cw) > g:
        pltpu.async_copy(eidx.at[c, pl.ds(base + g, g)], gidx.at[1], sem_i)
        pltpu.async_copy(eidx.at[1 - c, pl.ds(base + g, g)], sidx.at[1], sem_i)

    plsc.subcore_barrier()

    def step(i, _):
        grp = i // g
        pos = i % g
        slot = grp % 2
        buf = i % nb

        @pl.when(jnp.logical_and(pos == g - nb + 1, grp + 1 < ng))
        def _():
            nslot = (grp + 1) % 2
            pltpu.make_async_copy(eidx.at[c, pl.ds(base, g)],
                                  gidx.at[nslot], sem_i).wait()
            pltpu.make_async_copy(eidx.at[1 - c, pl.ds(base, g)],
                                  sidx.at[nslot], sem_i).wait()

        pltpu.make_async_copy(table.at[gidx.at[slot, pos]], rows.at[buf],
                              sem_g).wait()

        @pl.when(i < chunks - 1)
        def _():
            pltpu.async_copy(rows.at[buf], acc.at[sidx.at[slot, pos]], sem_s,
                             add=True)

        @pl.when(i == chunks - 1)
        def _():
            pltpu.sync_copy(rows.at[buf], acc.at[sidx.at[slot, pos]], add=True)

        @pl.when(i > 0)
        def _():
            pltpu.make_async_copy(
                rows.at[(i - 1) % nb],
                acc.at[sidx.at[((i - 1) // g) % 2, (i - 1) % g]], sem_s).wait()

        @pl.when(jnp.logical_and(pos == 0,
                                 jnp.logical_and(grp > 0, grp + 1 < ng)))
        def _():
            nbase = base + (grp + 1) * g
            pltpu.async_copy(eidx.at[c, pl.ds(nbase, g)],
                             gidx.at[(grp + 1) % 2], sem_i)
            pltpu.async_copy(eidx.at[1 - c, pl.ds(nbase, g)],
                             sidx.at[(grp + 1) % 2], sem_i)

        nxt = i + nb - 1

        @pl.when(jnp.logical_and(i > 0, nxt < chunks))
        def _():
            pltpu.async_copy(table.at[gidx.at[(nxt // g) % 2, nxt % g]],
                             rows.at[(i - 1) % nb], sem_g)
        return 0
    lax.fori_loop(0, chunks, step, 0)

    plsc.subcore_barrier()

    pltpu.sync_copy(acc.at[pl.ds(s * RPT, RPT)],
                    out.at[c, pl.ds(s * RPT, RPT)])


def _counts_body(eidx, out, acc, sidx, onesb, zbuf, sem_s, sem_i, *, cw, g):
    chunks = E // (NS * cw)
    ng = chunks // g
    c = lax.axis_index("c")
    s = lax.axis_index("s")
    base = s * chunks
    W = 8

    def fill(t, _):
        onesb[t, pl.ds(0, 16)] = jnp.ones((16,), jnp.float32)
        zbuf[t, pl.ds(0, 16)] = jnp.zeros((16,), jnp.float32)
        return 0
    lax.fori_loop(0, cw, fill, 0)
    for k in range(RPT // cw):
        pltpu.sync_copy(zbuf.at[pl.ds(0, cw)],
                        acc.at[pl.ds(s * RPT + k * cw, cw)])
    if RPT % cw:
        pltpu.sync_copy(zbuf.at[pl.ds(0, RPT % cw)],
                        acc.at[pl.ds(s * RPT + (RPT // cw) * cw, RPT % cw)])

    pltpu.sync_copy(eidx.at[1 - c, pl.ds(base, g)], sidx.at[0])
    if chunks > g:
        pltpu.async_copy(eidx.at[1 - c, pl.ds(base + g, g)], sidx.at[1], sem_i)

    plsc.subcore_barrier()

    def step(i, _):
        grp = i // g
        pos = i % g
        slot = grp % 3

        @pl.when(jnp.logical_and(pos == g - 1, grp + 1 < ng))
        def _():
            pltpu.make_async_copy(eidx.at[1 - c, pl.ds(base, g)],
                                  sidx.at[(grp + 1) % 3], sem_i).wait()

        pltpu.async_copy(onesb, acc.at[sidx.at[slot, pos]], sem_s, add=True)

        @pl.when(i >= W)
        def _():
            pltpu.make_async_copy(onesb, acc.at[sidx.at[slot, pos]],
                                  sem_s).wait()

        @pl.when(jnp.logical_and(pos == 0,
                                 jnp.logical_and(grp > 0, grp + 1 < ng)))
        def _():
            nbase = base + (grp + 1) * g
            pltpu.async_copy(eidx.at[1 - c, pl.ds(nbase, g)],
                             sidx.at[(grp + 1) % 3], sem_i)
        return 0
    lax.fori_loop(0, chunks, step, 0)
    for _ in range(W):
        pltpu.make_async_copy(onesb, acc.at[sidx.at[0, 0]], sem_s).wait()

    plsc.subcore_barrier()
    pltpu.sync_copy(acc.at[pl.ds(s * RPT, RPT)],
                    out.at[c, pl.ds(s * RPT, RPT)])


@jax.jit
def _counts(eidx):
    mesh = plsc.VectorSubcoreMesh(core_axis_name="c", subcore_axis_name="s",
                                  num_cores=NC, num_subcores=NS)
    f = pl.kernel(
        functools.partial(_counts_body, cw=CW, g=10),
        out_type=jax.ShapeDtypeStruct((2, N, 16), jnp.float32),
        mesh=mesh,
        scratch_types=[
            pltpu.VMEM_SHARED((N, 16), jnp.float32),
            pltpu.VMEM((3, 10, CW), jnp.int32),
            pltpu.VMEM((CW, 16), jnp.float32),
            pltpu.VMEM((CW, 16), jnp.float32),
            pltpu.SemaphoreType.DMA,
            pltpu.SemaphoreType.DMA,
        ],
        compiler_params=pltpu.CompilerParams(use_tc_tiling_on_sc=False),
    )
    return f(eidx)


@functools.partial(jax.jit, static_argnames=("dp", "nb", "cw", "g"))
def _segsum(table, eidx, *, dp, nb, cw, g):
    mesh = plsc.VectorSubcoreMesh(core_axis_name="c", subcore_axis_name="s",
                                  num_cores=NC, num_subcores=NS)
    f = pl.kernel(
        functools.partial(_segsum_body, dp=dp, nb=nb, cw=cw, g=g),
        out_type=jax.ShapeDtypeStruct((2, N, dp), jnp.float32),
        mesh=mesh,
        scratch_types=[
            pltpu.VMEM_SHARED((N, dp), jnp.float32),
            pltpu.VMEM((2, g, cw), jnp.int32),
            pltpu.VMEM((2, g, cw), jnp.int32),
            pltpu.VMEM((nb, cw, dp), jnp.float32),
            pltpu.SemaphoreType.DMA,
            pltpu.SemaphoreType.DMA,
            pltpu.SemaphoreType.DMA,
        ],
        compiler_params=pltpu.CompilerParams(use_tc_tiling_on_sc=False),
    )
    return f(table, eidx)


def _dense_body(sums_f, sums_b, cnt_f, cnt_b, xin, wl, wlt, wr, wrt, b, bt,
                out):
    sf = sums_f[0]
    sb = sums_b[0]
    mf = sf / jnp.maximum(cnt_f[0][:, 0:1], 1.0)
    mb = sb / jnp.maximum(cnt_b[0][:, 0:1], 1.0)
    xs = xin[...]
    w_self = wr[...] + wrt[...]
    out[...] = (jnp.dot(mf, wl[...], preferred_element_type=jnp.float32)
                + jnp.dot(mb, wlt[...], preferred_element_type=jnp.float32)
                + jnp.dot(xs, w_self, preferred_element_type=jnp.float32)
                + b[...] + bt[...])


@jax.jit
def _dense(sums, cnts, xin, wl, wlt, wr, wrt, b, bt):
    blk = 1000
    grid = (N // blk,)
    return pl.pallas_call(
        _dense_body,
        grid=grid,
        in_specs=[
            pl.BlockSpec((1, blk, D), lambda i: (0, i, 0)),
            pl.BlockSpec((1, blk, D), lambda i: (1, i, 0)),
            pl.BlockSpec((1, blk, 16), lambda i: (0, i, 0)),
            pl.BlockSpec((1, blk, 16), lambda i: (1, i, 0)),
            pl.BlockSpec((blk, D), lambda i: (i, 0)),
            pl.BlockSpec((D, D), lambda i: (0, 0)),
            pl.BlockSpec((D, D), lambda i: (0, 0)),
            pl.BlockSpec((D, D), lambda i: (0, 0)),
            pl.BlockSpec((D, D), lambda i: (0, 0)),
            pl.BlockSpec((1, D), lambda i: (0, 0)),
            pl.BlockSpec((1, D), lambda i: (0, 0)),
        ],
        out_specs=pl.BlockSpec((blk, D), lambda i: (i, 0)),
        out_shape=jax.ShapeDtypeStruct((N, D), jnp.float32),
    )(sums, sums, cnts, cnts, xin, wl, wlt, wr, wrt, b, bt)


def kernel(x, edge_index, Wl_1, Wr_1, b_1, Wl_1T, Wr_1T, b_1T,
           Wl_2, Wr_2, b_2, Wl_2T, Wr_2T, b_2T):
    ei = edge_index.astype(jnp.int32).reshape(2, E // CW, CW)
    b_1 = b_1.reshape(1, D)
    b_1T = b_1T.reshape(1, D)
    b_2 = b_2.reshape(1, D)
    b_2T = b_2T.reshape(1, D)

    cnts = _counts(ei)
    sums1 = _segsum(x, ei, dp=D, nb=4, cw=CW, g=10)
    h = _dense(sums1, cnts, x, Wl_1, Wl_1T, Wr_1, Wr_1T, b_1, b_1T)
    sums2 = _segsum(h, ei, dp=D, nb=4, cw=CW, g=10)
    out = _dense(sums2, cnts, h, Wl_2, Wl_2T, Wr_2, Wr_2T, b_2, b_2T)
    return out
